# Initial kernel scaffold; baseline (speedup 1.0000x reference)
import jax
import jax.numpy as jnp
from jax.experimental import pallas as pl


def kernel(x, edge_index, W0, b0, W1, b1, W2, b2):
    raise NotImplementedError("write your pallas kernel here")


# speedup vs baseline: 9.6278x; 9.6278x over previous
import functools

import jax
import jax.numpy as jnp
from jax import lax
from jax.experimental import pallas as pl
from jax.experimental.pallas import tpu as pltpu
from jax.experimental.pallas import tpu_sc as plsc

NC = 2
NS = 16
NW = NC * NS
CHUNK = 128
ROW_BLK = 1024
F32 = jnp.float32


def _sc_mesh():
    return plsc.VectorSubcoreMesh(
        core_axis_name="c", subcore_axis_name="s", num_cores=NC, num_subcores=NS
    )


@functools.lru_cache(maxsize=None)
def _make_deg_kernel(n_nodes, n_pad, e_pad):
    per_w = e_pad // NW
    iters = per_w // CHUNK
    rpt = n_pad // NS

    def body(src_hbm, dst_hbm, degp_hbm, dst2_hbm, sidx, didx, d2, ones, zbuf, acc):
        c = lax.axis_index("c")
        s = lax.axis_index("s")
        wid = s * NC + c
        for j in range(CHUNK // 16):
            ones[pl.ds(j * 16, 16)] = jnp.ones((16,), F32)
        for j in range(rpt // 16):
            zbuf[pl.ds(j * 16, 16)] = jnp.zeros((16,), F32)
        pltpu.sync_copy(zbuf, acc.at[pl.ds(s * rpt, rpt)])
        plsc.subcore_barrier()

        lane = lax.iota(jnp.int32, 16)

        def step(i, carry):
            off = wid * per_w + i * CHUNK
            pltpu.sync_copy(src_hbm.at[pl.ds(off, CHUNK)], sidx)
            pltpu.sync_copy(dst_hbm.at[pl.ds(off, CHUNK)], didx)
            for j in range(CHUNK // 16):
                sv = sidx[pl.ds(j * 16, 16)]
                dv = didx[pl.ds(j * 16, 16)]
                d2[pl.ds(j * 16, 16)] = jnp.where(sv == dv, n_nodes + lane, dv)
            pltpu.sync_copy(d2, dst2_hbm.at[pl.ds(off, CHUNK)])
            pltpu.sync_copy(ones, acc.at[d2], add=True)
            return carry

        lax.fori_loop(0, iters, step, 0)
        plsc.subcore_barrier()
        pltpu.sync_copy(acc.at[pl.ds(s * rpt, rpt)], degp_hbm.at[c, pl.ds(s * rpt, rpt)])

    return pl.kernel(
        body,
        out_type=(
            jax.ShapeDtypeStruct((NC, n_pad), F32),
            jax.ShapeDtypeStruct((e_pad,), jnp.int32),
        ),
        mesh=_sc_mesh(),
        scratch_types=[
            pltpu.VMEM((CHUNK,), jnp.int32),
            pltpu.VMEM((CHUNK,), jnp.int32),
            pltpu.VMEM((CHUNK,), jnp.int32),
            pltpu.VMEM((CHUNK,), F32),
            pltpu.VMEM((rpt,), F32),
            pltpu.VMEM_SHARED((n_pad,), F32),
        ],
    )


@functools.lru_cache(maxsize=None)
def _make_spmm_kernel(n_pad, e_pad, f):
    per_w = e_pad // NW
    iters = per_w // CHUNK
    rpt = n_pad // NS

    def body(y_hbm, src_hbm, dst2_hbm, out_hbm, sidx, didx, rows, zrow, acc, sem):
        c = lax.axis_index("c")
        s = lax.axis_index("s")
        wid = s * NC + c
        for r in range(16):
            for j in range(f // 16):
                zrow[r, pl.ds(j * 16, 16)] = jnp.zeros((16,), F32)

        def zstep(k, carry):
            pltpu.sync_copy(zrow, acc.at[pl.ds(s * rpt + k * 16, 16)])
            return carry

        lax.fori_loop(0, rpt // 16, zstep, 0)
        plsc.subcore_barrier()

        def step(i, carry):
            off = wid * per_w + i * CHUNK
            pltpu.sync_copy(src_hbm.at[pl.ds(off, CHUNK)], sidx)
            pltpu.sync_copy(dst2_hbm.at[pl.ds(off, CHUNK)], didx)
            pltpu.async_copy(y_hbm.at[sidx], rows, sem).wait()
            pltpu.sync_copy(rows, acc.at[didx], add=True)
            return carry

        lax.fori_loop(0, iters, step, 0)
        plsc.subcore_barrier()
        pltpu.sync_copy(acc.at[pl.ds(s * rpt, rpt)], out_hbm.at[c, pl.ds(s * rpt, rpt)])

    return pl.kernel(
        body,
        out_type=jax.ShapeDtypeStruct((NC, n_pad, f), F32),
        mesh=_sc_mesh(),
        scratch_types=[
            pltpu.VMEM((CHUNK,), jnp.int32),
            pltpu.VMEM((CHUNK,), jnp.int32),
            pltpu.VMEM((CHUNK, f), F32),
            pltpu.VMEM((16, f), F32),
            pltpu.VMEM_SHARED((n_pad, f), F32),
            pltpu.SemaphoreType.DMA,
        ],
    )


def _prep_body(degp_ref, x_ref, y_ref, dis_ref, sw_ref):
    deg = degp_ref[0] + degp_ref[1] + 1.0
    dis = lax.rsqrt(deg)
    dis_ref[...] = dis
    sw_ref[...] = 1.0 / deg
    y_ref[...] = x_ref[...] * dis


def _comb_body(p_ref, y_ref, sw_ref, o_ref):
    o_ref[...] = (p_ref[0] + p_ref[1] + y_ref[...]) * sw_ref[...]


def _run_prep(degp3, x_p, n_pad, f):
    grid = (n_pad // ROW_BLK,)
    return pl.pallas_call(
        _prep_body,
        grid=grid,
        in_specs=[
            pl.BlockSpec((NC, ROW_BLK, 1), lambda i: (0, i, 0)),
            pl.BlockSpec((ROW_BLK, f), lambda i: (i, 0)),
        ],
        out_specs=[
            pl.BlockSpec((ROW_BLK, f), lambda i: (i, 0)),
            pl.BlockSpec((ROW_BLK, 1), lambda i: (i, 0)),
            pl.BlockSpec((ROW_BLK, 1), lambda i: (i, 0)),
        ],
        out_shape=[
            jax.ShapeDtypeStruct((n_pad, f), F32),
            jax.ShapeDtypeStruct((n_pad, 1), F32),
            jax.ShapeDtypeStruct((n_pad, 1), F32),
        ],
    )(degp3, x_p)


def _run_comb(p, y, sw, n_pad, f):
    grid = (n_pad // ROW_BLK,)
    return pl.pallas_call(
        _comb_body,
        grid=grid,
        in_specs=[
            pl.BlockSpec((NC, ROW_BLK, f), lambda i: (0, i, 0)),
            pl.BlockSpec((ROW_BLK, f), lambda i: (i, 0)),
            pl.BlockSpec((ROW_BLK, 1), lambda i: (i, 0)),
        ],
        out_specs=pl.BlockSpec((ROW_BLK, f), lambda i: (i, 0)),
        out_shape=jax.ShapeDtypeStruct((n_pad, f), F32),
    )(p, y, sw)


def _make_mlp_body(n_classes):
    def body(p_ref, y_ref, dis_ref, w0_ref, b0_ref, w1_ref, b1_ref, w2_ref, b2_ref, o_ref):
        h = (p_ref[0] + p_ref[1] + y_ref[...]) * dis_ref[...]
        h = jnp.dot(h, w0_ref[...], preferred_element_type=F32) + b0_ref[...]
        h = jnp.maximum(h, 0.0)
        h = jnp.dot(h, w1_ref[...], preferred_element_type=F32) + b1_ref[...]
        h = jnp.maximum(h, 0.0)
        logits = jnp.dot(h, w2_ref[...], preferred_element_type=F32) + b2_ref[...]
        col = lax.broadcasted_iota(jnp.int32, logits.shape, 1)
        masked = jnp.where(col < n_classes, logits, jnp.float32(-1e30))
        m = jnp.max(masked, axis=1, keepdims=True)
        ssum = jnp.sum(jnp.exp(masked - m), axis=1, keepdims=True)
        o_ref[...] = logits - m - jnp.log(ssum)

    return body


def _run_mlp(p, y, dis, w0t, b0, w1t, b1, w2tp, b2p, n_pad, f, n_classes):
    h = w0t.shape[1]
    cp = w2tp.shape[1]
    grid = (n_pad // ROW_BLK,)
    return pl.pallas_call(
        _make_mlp_body(n_classes),
        grid=grid,
        in_specs=[
            pl.BlockSpec((NC, ROW_BLK, f), lambda i: (0, i, 0)),
            pl.BlockSpec((ROW_BLK, f), lambda i: (i, 0)),
            pl.BlockSpec((ROW_BLK, 1), lambda i: (i, 0)),
            pl.BlockSpec((f, h), lambda i: (0, 0)),
            pl.BlockSpec((h,), lambda i: (0,)),
            pl.BlockSpec((h, h), lambda i: (0, 0)),
            pl.BlockSpec((h,), lambda i: (0,)),
            pl.BlockSpec((h, cp), lambda i: (0, 0)),
            pl.BlockSpec((cp,), lambda i: (0,)),
        ],
        out_specs=pl.BlockSpec((ROW_BLK, cp), lambda i: (i, 0)),
        out_shape=jax.ShapeDtypeStruct((n_pad, cp), F32),
    )(p, y, dis, w0t, b0, w1t, b1, w2tp, b2p)


def kernel(x, edge_index, W0, b0, W1, b1, W2, b2):
    n_nodes, f = x.shape
    h_dim = W0.shape[0]
    n_classes = W2.shape[0]
    e = edge_index.shape[1]

    blk = NS * 16
    n_pad = -(-(n_nodes + 16) // blk) * blk
    n_pad = -(-n_pad // ROW_BLK) * ROW_BLK
    e_pad = -(-e // (NW * CHUNK)) * (NW * CHUNK)

    src = edge_index[0].astype(jnp.int32)
    dst = edge_index[1].astype(jnp.int32)
    npad_edges = e_pad - e
    ar = jnp.arange(npad_edges, dtype=jnp.int32)
    src_p = jnp.concatenate([src, (ar * 97) % n_nodes])
    dst_p = jnp.concatenate([dst, n_nodes + ar % (n_pad - n_nodes)])
    x_p = jnp.pad(x.astype(F32), ((0, n_pad - n_nodes), (0, 0)))

    degp, dst2 = _make_deg_kernel(n_nodes, n_pad, e_pad)(src_p, dst_p)
    y, dis, sw = _run_prep(degp.reshape(NC, n_pad, 1), x_p, n_pad, f)

    spmm = _make_spmm_kernel(n_pad, e_pad, f)
    for layer in range(3):
        p = spmm(y, src_p, dst2)
        if layer < 2:
            y = _run_comb(p, y, sw, n_pad, f)

    cp = 128
    w2tp = jnp.zeros((h_dim, cp), F32).at[:, :n_classes].set(W2.T.astype(F32))
    b2p = jnp.zeros((cp,), F32).at[:n_classes].set(b2.astype(F32))
    out = _run_mlp(p, y, dis, W0.T.astype(F32), b0.astype(F32),
                   W1.T.astype(F32), b1.astype(F32), w2tp, b2p,
                   n_pad, f, n_classes)
    return out[:n_nodes, :n_classes]


# speedup vs baseline: 14.4637x; 1.5023x over previous
import functools

import jax
import jax.numpy as jnp
from jax import lax
from jax.experimental import pallas as pl
from jax.experimental.pallas import tpu as pltpu
from jax.experimental.pallas import tpu_sc as plsc

NC = 2
NS = 16
NW = NC * NS
CHUNK = 128
ROW_BLK = 1024
F32 = jnp.float32


def _sc_mesh():
    return plsc.VectorSubcoreMesh(
        core_axis_name="c", subcore_axis_name="s", num_cores=NC, num_subcores=NS
    )


@functools.lru_cache(maxsize=None)
def _make_deg_kernel(n_nodes, n_pad, e_pad):
    per_w = e_pad // NW
    iters = per_w // CHUNK
    rpt = n_pad // NS

    assert iters % NBUF == 0
    groups = iters // NBUF

    def body(src_hbm, dst_hbm, degp_hbm, dst2_hbm,
             sidx, didx, d2, ones, zbuf, acc, *sems):
        isem_s = sems[0 * NBUF:1 * NBUF]
        isem_d = sems[1 * NBUF:2 * NBUF]
        osem = sems[2 * NBUF:3 * NBUF]
        ssem = sems[3 * NBUF:4 * NBUF]
        c = lax.axis_index("c")
        s = lax.axis_index("s")
        wid = s * NC + c
        for j in range(CHUNK // 16):
            ones[pl.ds(j * 16, 16)] = jnp.ones((16,), F32)
        for j in range(rpt // 16):
            zbuf[pl.ds(j * 16, 16)] = jnp.zeros((16,), F32)
        pltpu.sync_copy(zbuf, acc.at[pl.ds(s * rpt, rpt)])
        plsc.subcore_barrier()

        lane = lax.iota(jnp.int32, 16)

        def group(g, carry):
            base = wid * per_w + g * (NBUF * CHUNK)
            sdesc, ddesc = [], []
            for b in range(NBUF):

                @pl.when(g > 0)
                def _(b=b):
                    off0 = base + b * CHUNK
                    pltpu.make_async_copy(
                        d2.at[b], dst2_hbm.at[pl.ds(off0, CHUNK)],
                        osem[b]).wait()
                    pltpu.make_async_copy(
                        ones, acc.at[d2.at[b]], ssem[b]).wait()

                off = base + b * CHUNK
                sdesc.append(pltpu.async_copy(
                    src_hbm.at[pl.ds(off, CHUNK)], sidx.at[b], isem_s[b]))
                ddesc.append(pltpu.async_copy(
                    dst_hbm.at[pl.ds(off, CHUNK)], didx.at[b], isem_d[b]))
            for b in range(NBUF):
                sdesc[b].wait()
                ddesc[b].wait()
                for j in range(CHUNK // 16):
                    sv = sidx[b, pl.ds(j * 16, 16)]
                    dv = didx[b, pl.ds(j * 16, 16)]
                    d2[b, pl.ds(j * 16, 16)] = jnp.where(
                        sv == dv, n_nodes + lane, dv)
                off = base + b * CHUNK
                pltpu.async_copy(d2.at[b], dst2_hbm.at[pl.ds(off, CHUNK)],
                                 osem[b])
                pltpu.async_copy(ones, acc.at[d2.at[b]], ssem[b], add=True)
            return carry

        lax.fori_loop(0, groups, group, 0)
        for b in range(NBUF):
            pltpu.make_async_copy(
                d2.at[b], dst2_hbm.at[pl.ds(0, CHUNK)], osem[b]).wait()
            pltpu.make_async_copy(ones, acc.at[d2.at[b]], ssem[b]).wait()
        plsc.subcore_barrier()
        pltpu.sync_copy(acc.at[pl.ds(s * rpt, rpt)], degp_hbm.at[c, pl.ds(s * rpt, rpt)])

    return pl.kernel(
        body,
        out_type=(
            jax.ShapeDtypeStruct((NC, n_pad), F32),
            jax.ShapeDtypeStruct((e_pad,), jnp.int32),
        ),
        mesh=_sc_mesh(),
        scratch_types=[
            pltpu.VMEM((NBUF, CHUNK), jnp.int32),
            pltpu.VMEM((NBUF, CHUNK), jnp.int32),
            pltpu.VMEM((NBUF, CHUNK), jnp.int32),
            pltpu.VMEM((CHUNK,), F32),
            pltpu.VMEM((rpt,), F32),
            pltpu.VMEM_SHARED((n_pad,), F32),
        ] + [pltpu.SemaphoreType.DMA] * (4 * NBUF),
    )


NBUF = 2


@functools.lru_cache(maxsize=None)
def _make_spmm_kernel(n_pad, e_pad, f):
    per_w = e_pad // NW
    iters = per_w // CHUNK
    assert iters % NBUF == 0
    groups = iters // NBUF
    rpt = n_pad // NS

    zr = 16

    def body(y_hbm, src_hbm, dst2_hbm, out_hbm,
             sidx, didx, rows, zrow, acc, *sems):
        isem_s = sems[0 * NBUF:1 * NBUF]
        isem_d = sems[1 * NBUF:2 * NBUF]
        gsem = sems[2 * NBUF:3 * NBUF]
        ssem = sems[3 * NBUF:4 * NBUF]
        c = lax.axis_index("c")
        s = lax.axis_index("s")
        wid = s * NC + c
        for r in range(zr):
            for j in range(f // 16):
                zrow[r, pl.ds(j * 16, 16)] = jnp.zeros((16,), F32)

        def zstep(k, carry):
            pltpu.sync_copy(zrow, acc.at[pl.ds(s * rpt + k * zr, zr)])
            return carry

        lax.fori_loop(0, rpt // zr, zstep, 0)
        plsc.subcore_barrier()

        def group(g, carry):
            base = wid * per_w + g * (NBUF * CHUNK)
            sdesc, ddesc, gdesc = [], [], []
            for b in range(NBUF):

                @pl.when(g > 0)
                def _(b=b):
                    pltpu.make_async_copy(
                        rows.at[b], acc.at[didx.at[b]], ssem[b]).wait()

                off = base + b * CHUNK
                sdesc.append(pltpu.async_copy(
                    src_hbm.at[pl.ds(off, CHUNK)], sidx.at[b], isem_s[b]))
                ddesc.append(pltpu.async_copy(
                    dst2_hbm.at[pl.ds(off, CHUNK)], didx.at[b], isem_d[b]))
            for b in range(NBUF):
                sdesc[b].wait()
                gdesc.append(pltpu.async_copy(
                    y_hbm.at[sidx.at[b]], rows.at[b], gsem[b]))
            for b in range(NBUF):
                gdesc[b].wait()
                ddesc[b].wait()
                pltpu.async_copy(
                    rows.at[b], acc.at[didx.at[b]], ssem[b], add=True)
            return carry

        lax.fori_loop(0, groups, group, 0)
        for b in range(NBUF):
            pltpu.make_async_copy(
                rows.at[b], acc.at[didx.at[b]], ssem[b]).wait()
        plsc.subcore_barrier()
        pltpu.sync_copy(acc.at[pl.ds(s * rpt, rpt)], out_hbm.at[c, pl.ds(s * rpt, rpt)])

    return pl.kernel(
        body,
        out_type=jax.ShapeDtypeStruct((NC, n_pad, f), F32),
        mesh=_sc_mesh(),
        scratch_types=[
            pltpu.VMEM((NBUF, CHUNK), jnp.int32),
            pltpu.VMEM((NBUF, CHUNK), jnp.int32),
            pltpu.VMEM((NBUF, CHUNK, f), F32),
            pltpu.VMEM((zr, f), F32),
            pltpu.VMEM_SHARED((n_pad, f), F32),
        ] + [pltpu.SemaphoreType.DMA] * (4 * NBUF),
    )


def _prep_body(degp_ref, x_ref, y_ref, dis_ref, sw_ref):
    deg = degp_ref[0] + degp_ref[1] + 1.0
    dis = lax.rsqrt(deg)
    dis_ref[...] = dis
    sw_ref[...] = 1.0 / deg
    y_ref[...] = x_ref[...] * dis


def _comb_body(p_ref, y_ref, sw_ref, o_ref):
    o_ref[...] = (p_ref[0] + p_ref[1] + y_ref[...]) * sw_ref[...]


def _run_prep(degp3, x_p, n_pad, f):
    grid = (n_pad // ROW_BLK,)
    return pl.pallas_call(
        _prep_body,
        grid=grid,
        in_specs=[
            pl.BlockSpec((NC, ROW_BLK, 1), lambda i: (0, i, 0)),
            pl.BlockSpec((ROW_BLK, f), lambda i: (i, 0)),
        ],
        out_specs=[
            pl.BlockSpec((ROW_BLK, f), lambda i: (i, 0)),
            pl.BlockSpec((ROW_BLK, 1), lambda i: (i, 0)),
            pl.BlockSpec((ROW_BLK, 1), lambda i: (i, 0)),
        ],
        out_shape=[
            jax.ShapeDtypeStruct((n_pad, f), F32),
            jax.ShapeDtypeStruct((n_pad, 1), F32),
            jax.ShapeDtypeStruct((n_pad, 1), F32),
        ],
    )(degp3, x_p)


def _run_comb(p, y, sw, n_pad, f):
    grid = (n_pad // ROW_BLK,)
    return pl.pallas_call(
        _comb_body,
        grid=grid,
        in_specs=[
            pl.BlockSpec((NC, ROW_BLK, f), lambda i: (0, i, 0)),
            pl.BlockSpec((ROW_BLK, f), lambda i: (i, 0)),
            pl.BlockSpec((ROW_BLK, 1), lambda i: (i, 0)),
        ],
        out_specs=pl.BlockSpec((ROW_BLK, f), lambda i: (i, 0)),
        out_shape=jax.ShapeDtypeStruct((n_pad, f), F32),
    )(p, y, sw)


def _make_mlp_body(n_classes):
    def body(p_ref, y_ref, dis_ref, w0_ref, b0_ref, w1_ref, b1_ref, w2_ref, b2_ref, o_ref):
        h = (p_ref[0] + p_ref[1] + y_ref[...]) * dis_ref[...]
        h = jnp.dot(h, w0_ref[...], preferred_element_type=F32) + b0_ref[...]
        h = jnp.maximum(h, 0.0)
        h = jnp.dot(h, w1_ref[...], preferred_element_type=F32) + b1_ref[...]
        h = jnp.maximum(h, 0.0)
        logits = jnp.dot(h, w2_ref[...], preferred_element_type=F32) + b2_ref[...]
        col = lax.broadcasted_iota(jnp.int32, logits.shape, 1)
        masked = jnp.where(col < n_classes, logits, jnp.float32(-1e30))
        m = jnp.max(masked, axis=1, keepdims=True)
        ssum = jnp.sum(jnp.exp(masked - m), axis=1, keepdims=True)
        o_ref[...] = logits - m - jnp.log(ssum)

    return body


def _run_mlp(p, y, dis, w0t, b0, w1t, b1, w2tp, b2p, n_pad, f, n_classes):
    h = w0t.shape[1]
    cp = w2tp.shape[1]
    grid = (n_pad // ROW_BLK,)
    return pl.pallas_call(
        _make_mlp_body(n_classes),
        grid=grid,
        in_specs=[
            pl.BlockSpec((NC, ROW_BLK, f), lambda i: (0, i, 0)),
            pl.BlockSpec((ROW_BLK, f), lambda i: (i, 0)),
            pl.BlockSpec((ROW_BLK, 1), lambda i: (i, 0)),
            pl.BlockSpec((f, h), lambda i: (0, 0)),
            pl.BlockSpec((h,), lambda i: (0,)),
            pl.BlockSpec((h, h), lambda i: (0, 0)),
            pl.BlockSpec((h,), lambda i: (0,)),
            pl.BlockSpec((h, cp), lambda i: (0, 0)),
            pl.BlockSpec((cp,), lambda i: (0,)),
        ],
        out_specs=pl.BlockSpec((ROW_BLK, cp), lambda i: (i, 0)),
        out_shape=jax.ShapeDtypeStruct((n_pad, cp), F32),
    )(p, y, dis, w0t, b0, w1t, b1, w2tp, b2p)


def kernel(x, edge_index, W0, b0, W1, b1, W2, b2):
    n_nodes, f = x.shape
    h_dim = W0.shape[0]
    n_classes = W2.shape[0]
    e = edge_index.shape[1]

    blk = NS * 16
    n_pad = -(-(n_nodes + 16) // blk) * blk
    n_pad = -(-n_pad // ROW_BLK) * ROW_BLK
    e_pad = -(-e // (NW * CHUNK * NBUF)) * (NW * CHUNK * NBUF)

    src = edge_index[0].astype(jnp.int32)
    dst = edge_index[1].astype(jnp.int32)
    npad_edges = e_pad - e
    ar = jnp.arange(npad_edges, dtype=jnp.int32)
    src_p = jnp.concatenate([src, (ar * 97) % n_nodes])
    dst_p = jnp.concatenate([dst, n_nodes + ar % (n_pad - n_nodes)])
    x_p = jnp.pad(x.astype(F32), ((0, n_pad - n_nodes), (0, 0)))

    degp, dst2 = _make_deg_kernel(n_nodes, n_pad, e_pad)(src_p, dst_p)
    y, dis, sw = _run_prep(degp.reshape(NC, n_pad, 1), x_p, n_pad, f)

    spmm = _make_spmm_kernel(n_pad, e_pad, f)
    for layer in range(3):
        p = spmm(y, src_p, dst2)
        if layer < 2:
            y = _run_comb(p, y, sw, n_pad, f)

    cp = 128
    w2tp = jnp.zeros((h_dim, cp), F32).at[:, :n_classes].set(W2.T.astype(F32))
    b2p = jnp.zeros((cp,), F32).at[:n_classes].set(b2.astype(F32))
    out = _run_mlp(p, y, dis, W0.T.astype(F32), b0.astype(F32),
                   W1.T.astype(F32), b1.astype(F32), w2tp, b2p,
                   n_pad, f, n_classes)
    return out[:n_nodes, :n_classes]


# speedup vs baseline: 14.4843x; 1.0014x over previous
import functools

import jax
import jax.numpy as jnp
from jax import lax
from jax.experimental import pallas as pl
from jax.experimental.pallas import tpu as pltpu
from jax.experimental.pallas import tpu_sc as plsc

NC = 2
NS = 16
NW = NC * NS
CHUNK = 128
ROW_BLK = 1024
F32 = jnp.float32


def _sc_mesh():
    return plsc.VectorSubcoreMesh(
        core_axis_name="c", subcore_axis_name="s", num_cores=NC, num_subcores=NS
    )


@functools.lru_cache(maxsize=None)
def _make_deg_kernel(n_nodes, n_pad, e_pad):
    per_w = e_pad // NW
    iters = per_w // CHUNK
    rpt = n_pad // NS

    assert iters % NBUF == 0
    groups = iters // NBUF

    def body(src_hbm, dst_hbm, degp_hbm, pk_hbm,
             pkb, didx, ones, zbuf, acc, *sems):
        isem_s = sems[0 * NBUF:1 * NBUF]
        isem_d = sems[1 * NBUF:2 * NBUF]
        osem = sems[2 * NBUF:3 * NBUF]
        ssem = sems[3 * NBUF:4 * NBUF]
        c = lax.axis_index("c")
        s = lax.axis_index("s")
        wid = s * NC + c
        for j in range(CHUNK // 16):
            ones[pl.ds(j * 16, 16)] = jnp.ones((16,), F32)
        for j in range(rpt // 16):
            zbuf[pl.ds(j * 16, 16)] = jnp.zeros((16,), F32)
        pltpu.sync_copy(zbuf, acc.at[pl.ds(s * rpt, rpt)])
        plsc.subcore_barrier()

        lane = lax.iota(jnp.int32, 16)

        def group(g, carry):
            base = wid * per_w + g * (NBUF * CHUNK)
            cid0 = wid * iters + g * NBUF
            sdesc, ddesc = [], []
            for b in range(NBUF):

                @pl.when(g > 0)
                def _(b=b):
                    pltpu.make_async_copy(
                        pkb.at[b], pk_hbm.at[cid0 + b], osem[b]).wait()
                    pltpu.make_async_copy(
                        ones, acc.at[pkb.at[b, 1]], ssem[b]).wait()

                off = base + b * CHUNK
                sdesc.append(pltpu.async_copy(
                    src_hbm.at[pl.ds(off, CHUNK)], pkb.at[b, 0], isem_s[b]))
                ddesc.append(pltpu.async_copy(
                    dst_hbm.at[pl.ds(off, CHUNK)], didx.at[b], isem_d[b]))
            for b in range(NBUF):
                sdesc[b].wait()
                ddesc[b].wait()
                for j in range(CHUNK // 16):
                    sv = pkb[b, 0, pl.ds(j * 16, 16)]
                    dv = didx[b, pl.ds(j * 16, 16)]
                    pkb[b, 1, pl.ds(j * 16, 16)] = jnp.where(
                        sv == dv, n_nodes + lane, dv)
                pltpu.async_copy(pkb.at[b], pk_hbm.at[cid0 + b], osem[b])
                pltpu.async_copy(ones, acc.at[pkb.at[b, 1]], ssem[b], add=True)
            return carry

        lax.fori_loop(0, groups, group, 0)
        for b in range(NBUF):
            pltpu.make_async_copy(
                pkb.at[b], pk_hbm.at[0], osem[b]).wait()
            pltpu.make_async_copy(ones, acc.at[pkb.at[b, 1]], ssem[b]).wait()
        plsc.subcore_barrier()
        pltpu.sync_copy(acc.at[pl.ds(s * rpt, rpt)], degp_hbm.at[c, pl.ds(s * rpt, rpt)])

    return pl.kernel(
        body,
        out_type=(
            jax.ShapeDtypeStruct((NC, n_pad), F32),
            jax.ShapeDtypeStruct((e_pad // CHUNK, 2, CHUNK), jnp.int32),
        ),
        mesh=_sc_mesh(),
        scratch_types=[
            pltpu.VMEM((NBUF, 2, CHUNK), jnp.int32),
            pltpu.VMEM((NBUF, CHUNK), jnp.int32),
            pltpu.VMEM((CHUNK,), F32),
            pltpu.VMEM((rpt,), F32),
            pltpu.VMEM_SHARED((n_pad,), F32),
        ] + [pltpu.SemaphoreType.DMA] * (4 * NBUF),
    )


NBUF = 2


@functools.lru_cache(maxsize=None)
def _make_spmm_kernel(n_pad, e_pad, f):
    per_w = e_pad // NW
    iters = per_w // CHUNK
    assert iters % NBUF == 0
    groups = iters // NBUF
    rpt = n_pad // NS

    zr = 16

    def body(y_hbm, pk_hbm, out_hbm,
             pkb, rows, zrow, acc, *sems):
        isem = sems[0 * NBUF:1 * NBUF]
        gsem = sems[1 * NBUF:2 * NBUF]
        ssem = sems[2 * NBUF:3 * NBUF]
        c = lax.axis_index("c")
        s = lax.axis_index("s")
        wid = s * NC + c
        for r in range(zr):
            for j in range(f // 16):
                zrow[r, pl.ds(j * 16, 16)] = jnp.zeros((16,), F32)

        def zstep(k, carry):
            pltpu.sync_copy(zrow, acc.at[pl.ds(s * rpt + k * zr, zr)])
            return carry

        lax.fori_loop(0, rpt // zr, zstep, 0)
        plsc.subcore_barrier()

        def group(g, carry):
            cid0 = wid * iters + g * NBUF
            idesc, gdesc = [], []
            for b in range(NBUF):

                @pl.when(g > 0)
                def _(b=b):
                    pltpu.make_async_copy(
                        rows.at[b], acc.at[pkb.at[b, 1]], ssem[b]).wait()

                idesc.append(pltpu.async_copy(
                    pk_hbm.at[cid0 + b], pkb.at[b], isem[b]))
            for b in range(NBUF):
                idesc[b].wait()
                gdesc.append(pltpu.async_copy(
                    y_hbm.at[pkb.at[b, 0]], rows.at[b], gsem[b]))
            for b in range(NBUF):
                gdesc[b].wait()
                pltpu.async_copy(
                    rows.at[b], acc.at[pkb.at[b, 1]], ssem[b], add=True)
            return carry

        lax.fori_loop(0, groups, group, 0)
        for b in range(NBUF):
            pltpu.make_async_copy(
                rows.at[b], acc.at[pkb.at[b, 1]], ssem[b]).wait()
        plsc.subcore_barrier()
        pltpu.sync_copy(acc.at[pl.ds(s * rpt, rpt)], out_hbm.at[c, pl.ds(s * rpt, rpt)])

    return pl.kernel(
        body,
        out_type=jax.ShapeDtypeStruct((NC, n_pad, f), F32),
        mesh=_sc_mesh(),
        scratch_types=[
            pltpu.VMEM((NBUF, 2, CHUNK), jnp.int32),
            pltpu.VMEM((NBUF, CHUNK, f), F32),
            pltpu.VMEM((zr, f), F32),
            pltpu.VMEM_SHARED((n_pad, f), F32),
        ] + [pltpu.SemaphoreType.DMA] * (3 * NBUF),
    )


def _prep_body(degp_ref, x_ref, y_ref, dis_ref, sw_ref):
    deg = degp_ref[0] + degp_ref[1] + 1.0
    dis = lax.rsqrt(deg)
    dis_ref[...] = dis
    sw_ref[...] = 1.0 / deg
    y_ref[...] = x_ref[...] * dis


def _comb_body(p_ref, y_ref, sw_ref, o_ref):
    o_ref[...] = (p_ref[0] + p_ref[1] + y_ref[...]) * sw_ref[...]


def _run_prep(degp3, x_p, n_pad, f):
    grid = (n_pad // ROW_BLK,)
    return pl.pallas_call(
        _prep_body,
        grid=grid,
        in_specs=[
            pl.BlockSpec((NC, ROW_BLK, 1), lambda i: (0, i, 0)),
            pl.BlockSpec((ROW_BLK, f), lambda i: (i, 0)),
        ],
        out_specs=[
            pl.BlockSpec((ROW_BLK, f), lambda i: (i, 0)),
            pl.BlockSpec((ROW_BLK, 1), lambda i: (i, 0)),
            pl.BlockSpec((ROW_BLK, 1), lambda i: (i, 0)),
        ],
        out_shape=[
            jax.ShapeDtypeStruct((n_pad, f), F32),
            jax.ShapeDtypeStruct((n_pad, 1), F32),
            jax.ShapeDtypeStruct((n_pad, 1), F32),
        ],
    )(degp3, x_p)


def _run_comb(p, y, sw, n_pad, f):
    grid = (n_pad // ROW_BLK,)
    return pl.pallas_call(
        _comb_body,
        grid=grid,
        in_specs=[
            pl.BlockSpec((NC, ROW_BLK, f), lambda i: (0, i, 0)),
            pl.BlockSpec((ROW_BLK, f), lambda i: (i, 0)),
            pl.BlockSpec((ROW_BLK, 1), lambda i: (i, 0)),
        ],
        out_specs=pl.BlockSpec((ROW_BLK, f), lambda i: (i, 0)),
        out_shape=jax.ShapeDtypeStruct((n_pad, f), F32),
    )(p, y, sw)


def _make_mlp_body(n_classes):
    def body(p_ref, y_ref, dis_ref, w0_ref, b0_ref, w1_ref, b1_ref, w2_ref, b2_ref, o_ref):
        h = (p_ref[0] + p_ref[1] + y_ref[...]) * dis_ref[...]
        h = jnp.dot(h, w0_ref[...], preferred_element_type=F32) + b0_ref[...]
        h = jnp.maximum(h, 0.0)
        h = jnp.dot(h, w1_ref[...], preferred_element_type=F32) + b1_ref[...]
        h = jnp.maximum(h, 0.0)
        logits = jnp.dot(h, w2_ref[...], preferred_element_type=F32) + b2_ref[...]
        col = lax.broadcasted_iota(jnp.int32, logits.shape, 1)
        masked = jnp.where(col < n_classes, logits, jnp.float32(-1e30))
        m = jnp.max(masked, axis=1, keepdims=True)
        ssum = jnp.sum(jnp.exp(masked - m), axis=1, keepdims=True)
        o_ref[...] = logits - m - jnp.log(ssum)

    return body


def _run_mlp(p, y, dis, w0t, b0, w1t, b1, w2tp, b2p, n_pad, f, n_classes):
    h = w0t.shape[1]
    cp = w2tp.shape[1]
    grid = (n_pad // ROW_BLK,)
    return pl.pallas_call(
        _make_mlp_body(n_classes),
        grid=grid,
        in_specs=[
            pl.BlockSpec((NC, ROW_BLK, f), lambda i: (0, i, 0)),
            pl.BlockSpec((ROW_BLK, f), lambda i: (i, 0)),
            pl.BlockSpec((ROW_BLK, 1), lambda i: (i, 0)),
            pl.BlockSpec((f, h), lambda i: (0, 0)),
            pl.BlockSpec((h,), lambda i: (0,)),
            pl.BlockSpec((h, h), lambda i: (0, 0)),
            pl.BlockSpec((h,), lambda i: (0,)),
            pl.BlockSpec((h, cp), lambda i: (0, 0)),
            pl.BlockSpec((cp,), lambda i: (0,)),
        ],
        out_specs=pl.BlockSpec((ROW_BLK, cp), lambda i: (i, 0)),
        out_shape=jax.ShapeDtypeStruct((n_pad, cp), F32),
    )(p, y, dis, w0t, b0, w1t, b1, w2tp, b2p)


def kernel(x, edge_index, W0, b0, W1, b1, W2, b2):
    n_nodes, f = x.shape
    h_dim = W0.shape[0]
    n_classes = W2.shape[0]
    e = edge_index.shape[1]

    blk = NS * 16
    n_pad = -(-(n_nodes + 16) // blk) * blk
    n_pad = -(-n_pad // ROW_BLK) * ROW_BLK
    e_pad = -(-e // (NW * CHUNK * NBUF)) * (NW * CHUNK * NBUF)

    src = edge_index[0].astype(jnp.int32)
    dst = edge_index[1].astype(jnp.int32)
    npad_edges = e_pad - e
    ar = jnp.arange(npad_edges, dtype=jnp.int32)
    src_p = jnp.concatenate([src, (ar * 97) % n_nodes])
    dst_p = jnp.concatenate([dst, n_nodes + ar % (n_pad - n_nodes)])
    x_p = jnp.pad(x.astype(F32), ((0, n_pad - n_nodes), (0, 0)))

    degp, pk = _make_deg_kernel(n_nodes, n_pad, e_pad)(src_p, dst_p)
    y, dis, sw = _run_prep(degp.reshape(NC, n_pad, 1), x_p, n_pad, f)

    spmm = _make_spmm_kernel(n_pad, e_pad, f)
    for layer in range(3):
        p = spmm(y, pk)
        if layer < 2:
            y = _run_comb(p, y, sw, n_pad, f)

    cp = 128
    w2tp = jnp.zeros((h_dim, cp), F32).at[:, :n_classes].set(W2.T.astype(F32))
    b2p = jnp.zeros((cp,), F32).at[:n_classes].set(b2.astype(F32))
    out = _run_mlp(p, y, dis, W0.T.astype(F32), b0.astype(F32),
                   W1.T.astype(F32), b1.astype(F32), w2tp, b2p,
                   n_pad, f, n_classes)
    return out[:n_nodes, :n_classes]


# speedup vs baseline: 16.3972x; 1.1321x over previous
import functools

import jax
import jax.numpy as jnp
from jax import lax
from jax.experimental import pallas as pl
from jax.experimental.pallas import tpu as pltpu
from jax.experimental.pallas import tpu_sc as plsc

NC = 2
NS = 16
NW = NC * NS
CHUNK = 64
ROW_BLK = 1024
F32 = jnp.float32


def _sc_mesh():
    return plsc.VectorSubcoreMesh(
        core_axis_name="c", subcore_axis_name="s", num_cores=NC, num_subcores=NS
    )


@functools.lru_cache(maxsize=None)
def _make_deg_kernel(n_nodes, n_pad, e_pad):
    per_w = e_pad // NW
    iters = per_w // CHUNK
    rpt = n_pad // NS

    assert iters % NBUF == 0
    groups = iters // NBUF

    def body(src_hbm, dst_hbm, degp_hbm, pk_hbm,
             pkb, didx, ones, zbuf, acc, *sems):
        isem_s = sems[0 * NBUF:1 * NBUF]
        isem_d = sems[1 * NBUF:2 * NBUF]
        osem = sems[2 * NBUF:3 * NBUF]
        ssem = sems[3 * NBUF:4 * NBUF]
        c = lax.axis_index("c")
        s = lax.axis_index("s")
        wid = s * NC + c
        for j in range(CHUNK // 16):
            ones[pl.ds(j * 16, 16)] = jnp.ones((16,), F32)
        for j in range(rpt // 16):
            zbuf[pl.ds(j * 16, 16)] = jnp.zeros((16,), F32)
        pltpu.sync_copy(zbuf, acc.at[pl.ds(s * rpt, rpt)])
        plsc.subcore_barrier()

        lane = lax.iota(jnp.int32, 16)

        def group(g, carry):
            base = wid * per_w + g * (NBUF * CHUNK)
            cid0 = wid * iters + g * NBUF
            sdesc, ddesc = [], []
            for b in range(NBUF):

                @pl.when(g > 0)
                def _(b=b):
                    pltpu.make_async_copy(
                        pkb.at[b], pk_hbm.at[cid0 + b], osem[b]).wait()
                    pltpu.make_async_copy(
                        ones, acc.at[pkb.at[b, 1]], ssem[b]).wait()

                off = base + b * CHUNK
                sdesc.append(pltpu.async_copy(
                    src_hbm.at[pl.ds(off, CHUNK)], pkb.at[b, 0], isem_s[b]))
                ddesc.append(pltpu.async_copy(
                    dst_hbm.at[pl.ds(off, CHUNK)], didx.at[b], isem_d[b]))
            for b in range(NBUF):
                sdesc[b].wait()
                ddesc[b].wait()
                for j in range(CHUNK // 16):
                    sv = pkb[b, 0, pl.ds(j * 16, 16)]
                    dv = didx[b, pl.ds(j * 16, 16)]
                    pkb[b, 1, pl.ds(j * 16, 16)] = jnp.where(
                        sv == dv, n_nodes + lane, dv)
                pltpu.async_copy(pkb.at[b], pk_hbm.at[cid0 + b], osem[b])
                pltpu.async_copy(ones, acc.at[pkb.at[b, 1]], ssem[b], add=True)
            return carry

        lax.fori_loop(0, groups, group, 0)
        for b in range(NBUF):
            pltpu.make_async_copy(
                pkb.at[b], pk_hbm.at[0], osem[b]).wait()
            pltpu.make_async_copy(ones, acc.at[pkb.at[b, 1]], ssem[b]).wait()
        plsc.subcore_barrier()
        pltpu.sync_copy(acc.at[pl.ds(s * rpt, rpt)], degp_hbm.at[c, pl.ds(s * rpt, rpt)])

    return pl.kernel(
        body,
        out_type=(
            jax.ShapeDtypeStruct((NC, n_pad), F32),
            jax.ShapeDtypeStruct((e_pad // CHUNK, 2, CHUNK), jnp.int32),
        ),
        mesh=_sc_mesh(),
        scratch_types=[
            pltpu.VMEM((NBUF, 2, CHUNK), jnp.int32),
            pltpu.VMEM((NBUF, CHUNK), jnp.int32),
            pltpu.VMEM((CHUNK,), F32),
            pltpu.VMEM((rpt,), F32),
            pltpu.VMEM_SHARED((n_pad,), F32),
        ] + [pltpu.SemaphoreType.DMA] * (4 * NBUF),
    )


NBUF = 5


@functools.lru_cache(maxsize=None)
def _make_spmm_kernel(n_pad, e_pad, f):
    per_w = e_pad // NW
    iters = per_w // CHUNK
    assert iters % NBUF == 0
    groups = iters // NBUF
    rpt = n_pad // NS

    zr = 32
    assert rpt % zr == 0

    def body(y_hbm, pk_hbm, out_hbm,
             pkb2, rows, zrow, acc, *sems):
        isem = sems[0 * NBUF:2 * NBUF]
        gsem = sems[2 * NBUF:3 * NBUF]
        ssem = sems[3 * NBUF:5 * NBUF]
        zsem = sems[5 * NBUF]
        c = lax.axis_index("c")
        s = lax.axis_index("s")
        wid = s * NC + c
        for r in range(zr):
            for j in range(f // 16):
                zrow[r, pl.ds(j * 16, 16)] = jnp.zeros((16,), F32)

        def zstep(k, carry):
            pltpu.async_copy(zrow, acc.at[pl.ds(s * rpt + k * zr, zr)], zsem)
            return carry

        lax.fori_loop(0, rpt // zr, zstep, 0)

        def zdrain(k, carry):
            pltpu.make_async_copy(
                zrow, acc.at[pl.ds(s * rpt, zr)], zsem).wait()
            return carry

        lax.fori_loop(0, rpt // zr, zdrain, 0)
        plsc.subcore_barrier()

        assert groups % 2 == 0
        for b in range(NBUF):
            pltpu.async_copy(pk_hbm.at[wid * iters + b], pkb2.at[0, b],
                             isem[b])

        def pair(tt, carry):
            for par in range(2):
                it = tt * 2 + par
                cid_next = wid * iters + (it + 1) * NBUF
                gdesc = []
                for b in range(NBUF):

                    @pl.when(it > 0)
                    def _(b=b, par=par):
                        pltpu.make_async_copy(
                            rows.at[b], acc.at[pkb2.at[1 - par, b, 1]],
                            ssem[(1 - par) * NBUF + b]).wait()

                    @pl.when(it + 1 < groups)
                    def _(b=b, par=par, cid_next=cid_next):
                        pltpu.async_copy(
                            pk_hbm.at[cid_next + b], pkb2.at[1 - par, b],
                            isem[(1 - par) * NBUF + b])

                for b in range(NBUF):
                    pltpu.make_async_copy(
                        pk_hbm.at[wid * iters + b], pkb2.at[par, b],
                        isem[par * NBUF + b]).wait()
                    gdesc.append(pltpu.async_copy(
                        y_hbm.at[pkb2.at[par, b, 0]], rows.at[b], gsem[b]))
                for b in range(NBUF):
                    gdesc[b].wait()
                    pltpu.async_copy(
                        rows.at[b], acc.at[pkb2.at[par, b, 1]],
                        ssem[par * NBUF + b], add=True)
            return carry

        lax.fori_loop(0, groups // 2, pair, 0)
        lastpar = (groups - 1) % 2
        for b in range(NBUF):
            pltpu.make_async_copy(
                rows.at[b], acc.at[pkb2.at[lastpar, b, 1]],
                ssem[lastpar * NBUF + b]).wait()
        plsc.subcore_barrier()
        pltpu.sync_copy(acc.at[pl.ds(s * rpt, rpt)], out_hbm.at[c, pl.ds(s * rpt, rpt)])

    return pl.kernel(
        body,
        out_type=jax.ShapeDtypeStruct((NC, n_pad, f), F32),
        mesh=_sc_mesh(),
        scratch_types=[
            pltpu.VMEM((2, NBUF, 2, CHUNK), jnp.int32),
            pltpu.VMEM((NBUF, CHUNK, f), F32),
            pltpu.VMEM((zr, f), F32),
            pltpu.VMEM_SHARED((n_pad, f), F32),
        ] + [pltpu.SemaphoreType.DMA] * (5 * NBUF + 1),
    )


def _prep_body(degp_ref, x_ref, y_ref, dis_ref, sw_ref):
    deg = degp_ref[0] + degp_ref[1] + 1.0
    dis = lax.rsqrt(deg)
    dis_ref[...] = dis
    sw_ref[...] = 1.0 / deg
    y_ref[...] = x_ref[...] * dis


def _comb_body(p_ref, y_ref, sw_ref, o_ref):
    o_ref[...] = (p_ref[0] + p_ref[1] + y_ref[...]) * sw_ref[...]


def _run_prep(degp3, x_p, n_pad, f):
    grid = (n_pad // ROW_BLK,)
    return pl.pallas_call(
        _prep_body,
        grid=grid,
        in_specs=[
            pl.BlockSpec((NC, ROW_BLK, 1), lambda i: (0, i, 0)),
            pl.BlockSpec((ROW_BLK, f), lambda i: (i, 0)),
        ],
        out_specs=[
            pl.BlockSpec((ROW_BLK, f), lambda i: (i, 0)),
            pl.BlockSpec((ROW_BLK, 1), lambda i: (i, 0)),
            pl.BlockSpec((ROW_BLK, 1), lambda i: (i, 0)),
        ],
        out_shape=[
            jax.ShapeDtypeStruct((n_pad, f), F32),
            jax.ShapeDtypeStruct((n_pad, 1), F32),
            jax.ShapeDtypeStruct((n_pad, 1), F32),
        ],
    )(degp3, x_p)


def _run_comb(p, y, sw, n_pad, f):
    grid = (n_pad // ROW_BLK,)
    return pl.pallas_call(
        _comb_body,
        grid=grid,
        in_specs=[
            pl.BlockSpec((NC, ROW_BLK, f), lambda i: (0, i, 0)),
            pl.BlockSpec((ROW_BLK, f), lambda i: (i, 0)),
            pl.BlockSpec((ROW_BLK, 1), lambda i: (i, 0)),
        ],
        out_specs=pl.BlockSpec((ROW_BLK, f), lambda i: (i, 0)),
        out_shape=jax.ShapeDtypeStruct((n_pad, f), F32),
    )(p, y, sw)


def _make_mlp_body(n_classes):
    def body(p_ref, y_ref, dis_ref, w0_ref, b0_ref, w1_ref, b1_ref, w2_ref, b2_ref, o_ref):
        h = (p_ref[0] + p_ref[1] + y_ref[...]) * dis_ref[...]
        h = jnp.dot(h, w0_ref[...], preferred_element_type=F32) + b0_ref[...]
        h = jnp.maximum(h, 0.0)
        h = jnp.dot(h, w1_ref[...], preferred_element_type=F32) + b1_ref[...]
        h = jnp.maximum(h, 0.0)
        logits = jnp.dot(h, w2_ref[...], preferred_element_type=F32) + b2_ref[...]
        col = lax.broadcasted_iota(jnp.int32, logits.shape, 1)
        masked = jnp.where(col < n_classes, logits, jnp.float32(-1e30))
        m = jnp.max(masked, axis=1, keepdims=True)
        ssum = jnp.sum(jnp.exp(masked - m), axis=1, keepdims=True)
        o_ref[...] = logits - m - jnp.log(ssum)

    return body


def _run_mlp(p, y, dis, w0t, b0, w1t, b1, w2tp, b2p, n_pad, f, n_classes):
    h = w0t.shape[1]
    cp = w2tp.shape[1]
    grid = (n_pad // ROW_BLK,)
    return pl.pallas_call(
        _make_mlp_body(n_classes),
        grid=grid,
        in_specs=[
            pl.BlockSpec((NC, ROW_BLK, f), lambda i: (0, i, 0)),
            pl.BlockSpec((ROW_BLK, f), lambda i: (i, 0)),
            pl.BlockSpec((ROW_BLK, 1), lambda i: (i, 0)),
            pl.BlockSpec((f, h), lambda i: (0, 0)),
            pl.BlockSpec((h,), lambda i: (0,)),
            pl.BlockSpec((h, h), lambda i: (0, 0)),
            pl.BlockSpec((h,), lambda i: (0,)),
            pl.BlockSpec((h, cp), lambda i: (0, 0)),
            pl.BlockSpec((cp,), lambda i: (0,)),
        ],
        out_specs=pl.BlockSpec((ROW_BLK, cp), lambda i: (i, 0)),
        out_shape=jax.ShapeDtypeStruct((n_pad, cp), F32),
    )(p, y, dis, w0t, b0, w1t, b1, w2tp, b2p)


def kernel(x, edge_index, W0, b0, W1, b1, W2, b2):
    n_nodes, f = x.shape
    h_dim = W0.shape[0]
    n_classes = W2.shape[0]
    e = edge_index.shape[1]

    blk = NS * 16
    n_pad = -(-(n_nodes + 16) // blk) * blk
    n_pad = -(-n_pad // ROW_BLK) * ROW_BLK
    e_pad = -(-e // (NW * CHUNK * NBUF)) * (NW * CHUNK * NBUF)

    src = edge_index[0].astype(jnp.int32)
    dst = edge_index[1].astype(jnp.int32)
    npad_edges = e_pad - e
    ar = jnp.arange(npad_edges, dtype=jnp.int32)
    src_p = jnp.concatenate([src, (ar * 97) % n_nodes])
    dst_p = jnp.concatenate([dst, n_nodes + ar % (n_pad - n_nodes)])

    degp, pk = _make_deg_kernel(n_nodes, n_pad, e_pad)(src_p, dst_p)
    y, dis, sw = _run_prep(degp.reshape(NC, n_pad, 1), x.astype(F32),
                           n_pad, f)

    spmm = _make_spmm_kernel(n_pad, e_pad, f)
    for layer in range(3):
        p = spmm(y, pk)
        if layer < 2:
            y = _run_comb(p, y, sw, n_pad, f)

    cp = 128
    w2tp = jnp.zeros((h_dim, cp), F32).at[:, :n_classes].set(W2.T.astype(F32))
    b2p = jnp.zeros((cp,), F32).at[:n_classes].set(b2.astype(F32))
    out = _run_mlp(p, y, dis, W0.T.astype(F32), b0.astype(F32),
                   W1.T.astype(F32), b1.astype(F32), w2tp, b2p,
                   n_pad, f, n_classes)
    return out[:n_nodes, :n_classes]


# speedup vs baseline: 16.6594x; 1.0160x over previous
import functools

import jax
import jax.numpy as jnp
from jax import lax
from jax.experimental import pallas as pl
from jax.experimental.pallas import tpu as pltpu
from jax.experimental.pallas import tpu_sc as plsc

NC = 2
NS = 16
NW = NC * NS
CHUNK = 64
ROW_BLK = 2048
F32 = jnp.float32


def _sc_mesh():
    return plsc.VectorSubcoreMesh(
        core_axis_name="c", subcore_axis_name="s", num_cores=NC, num_subcores=NS
    )


@functools.lru_cache(maxsize=None)
def _make_deg_kernel(n_nodes, n_pad, e_pad):
    per_w = e_pad // NW
    iters = per_w // CHUNK
    rpt = n_pad // NS

    assert iters % NBUF == 0
    groups = iters // NBUF

    def body(src_hbm, dst_hbm, degp_hbm, pk_hbm,
             pkb, didx, ones, zbuf, acc, *sems):
        isem_s = sems[0 * NBUF:1 * NBUF]
        isem_d = sems[1 * NBUF:2 * NBUF]
        osem = sems[2 * NBUF:3 * NBUF]
        ssem = sems[3 * NBUF:4 * NBUF]
        c = lax.axis_index("c")
        s = lax.axis_index("s")
        wid = s * NC + c
        for j in range(CHUNK // 16):
            ones[pl.ds(j * 16, 16)] = jnp.ones((16,), F32)
        for j in range(rpt // 16):
            zbuf[pl.ds(j * 16, 16)] = jnp.zeros((16,), F32)
        pltpu.sync_copy(zbuf, acc.at[pl.ds(s * rpt, rpt)])
        plsc.subcore_barrier()

        lane = lax.iota(jnp.int32, 16)

        def group(g, carry):
            base = wid * per_w + g * (NBUF * CHUNK)
            cid0 = wid * iters + g * NBUF
            sdesc, ddesc = [], []
            for b in range(NBUF):

                @pl.when(g > 0)
                def _(b=b):
                    pltpu.make_async_copy(
                        pkb.at[b], pk_hbm.at[cid0 + b], osem[b]).wait()
                    pltpu.make_async_copy(
                        ones, acc.at[pkb.at[b, 1]], ssem[b]).wait()

                off = base + b * CHUNK
                sdesc.append(pltpu.async_copy(
                    src_hbm.at[pl.ds(off, CHUNK)], pkb.at[b, 0], isem_s[b]))
                ddesc.append(pltpu.async_copy(
                    dst_hbm.at[pl.ds(off, CHUNK)], didx.at[b], isem_d[b]))
            for b in range(NBUF):
                sdesc[b].wait()
                ddesc[b].wait()
                for j in range(CHUNK // 16):
                    sv = pkb[b, 0, pl.ds(j * 16, 16)]
                    dv = didx[b, pl.ds(j * 16, 16)]
                    pkb[b, 1, pl.ds(j * 16, 16)] = jnp.where(
                        sv == dv, n_nodes + lane, dv)
                pltpu.async_copy(pkb.at[b], pk_hbm.at[cid0 + b], osem[b])
                pltpu.async_copy(ones, acc.at[pkb.at[b, 1]], ssem[b], add=True)
            return carry

        lax.fori_loop(0, groups, group, 0)
        for b in range(NBUF):
            pltpu.make_async_copy(
                pkb.at[b], pk_hbm.at[0], osem[b]).wait()
            pltpu.make_async_copy(ones, acc.at[pkb.at[b, 1]], ssem[b]).wait()
        plsc.subcore_barrier()
        pltpu.sync_copy(acc.at[pl.ds(s * rpt, rpt)], degp_hbm.at[c, pl.ds(s * rpt, rpt)])

    return pl.kernel(
        body,
        out_type=(
            jax.ShapeDtypeStruct((NC, n_pad), F32),
            jax.ShapeDtypeStruct((e_pad // CHUNK, 2, CHUNK), jnp.int32),
        ),
        mesh=_sc_mesh(),
        scratch_types=[
            pltpu.VMEM((NBUF, 2, CHUNK), jnp.int32),
            pltpu.VMEM((NBUF, CHUNK), jnp.int32),
            pltpu.VMEM((CHUNK,), F32),
            pltpu.VMEM((rpt,), F32),
            pltpu.VMEM_SHARED((n_pad,), F32),
        ] + [pltpu.SemaphoreType.DMA] * (4 * NBUF),
    )


NBUF = 5


@functools.lru_cache(maxsize=None)
def _make_spmm_kernel(n_pad, e_pad, f):
    per_w = e_pad // NW
    iters = per_w // CHUNK
    assert iters % NBUF == 0
    groups = iters // NBUF
    rpt = n_pad // NS

    zr = 32
    assert rpt % zr == 0

    def body(y_hbm, pk_hbm, out_hbm,
             pkb2, rows, zrow, acc, *sems):
        isem = sems[0 * NBUF:2 * NBUF]
        gsem = sems[2 * NBUF:3 * NBUF]
        ssem = sems[3 * NBUF:5 * NBUF]
        zsem = sems[5 * NBUF]
        c = lax.axis_index("c")
        s = lax.axis_index("s")
        wid = s * NC + c
        for r in range(zr):
            for j in range(f // 16):
                zrow[r, pl.ds(j * 16, 16)] = jnp.zeros((16,), F32)

        def zstep(k, carry):
            pltpu.async_copy(zrow, acc.at[pl.ds(s * rpt + k * zr, zr)], zsem)
            return carry

        lax.fori_loop(0, rpt // zr, zstep, 0)

        def zdrain(k, carry):
            pltpu.make_async_copy(
                zrow, acc.at[pl.ds(s * rpt, zr)], zsem).wait()
            return carry

        lax.fori_loop(0, rpt // zr, zdrain, 0)
        plsc.subcore_barrier()

        assert groups % 2 == 0
        for b in range(NBUF):
            pltpu.async_copy(pk_hbm.at[wid * iters + b], pkb2.at[0, b],
                             isem[b])

        def pair(tt, carry):
            for par in range(2):
                it = tt * 2 + par
                cid_next = wid * iters + (it + 1) * NBUF
                gdesc = []
                for b in range(NBUF):

                    @pl.when(it > 0)
                    def _(b=b, par=par):
                        pltpu.make_async_copy(
                            rows.at[b], acc.at[pkb2.at[1 - par, b, 1]],
                            ssem[(1 - par) * NBUF + b]).wait()

                    @pl.when(it + 1 < groups)
                    def _(b=b, par=par, cid_next=cid_next):
                        pltpu.async_copy(
                            pk_hbm.at[cid_next + b], pkb2.at[1 - par, b],
                            isem[(1 - par) * NBUF + b])

                for b in range(NBUF):
                    pltpu.make_async_copy(
                        pk_hbm.at[wid * iters + b], pkb2.at[par, b],
                        isem[par * NBUF + b]).wait()
                    gdesc.append(pltpu.async_copy(
                        y_hbm.at[pkb2.at[par, b, 0]], rows.at[b], gsem[b]))
                for b in range(NBUF):
                    gdesc[b].wait()
                    pltpu.async_copy(
                        rows.at[b], acc.at[pkb2.at[par, b, 1]],
                        ssem[par * NBUF + b], add=True)
            return carry

        lax.fori_loop(0, groups // 2, pair, 0)
        lastpar = (groups - 1) % 2
        for b in range(NBUF):
            pltpu.make_async_copy(
                rows.at[b], acc.at[pkb2.at[lastpar, b, 1]],
                ssem[lastpar * NBUF + b]).wait()
        plsc.subcore_barrier()
        pltpu.sync_copy(acc.at[pl.ds(s * rpt, rpt)], out_hbm.at[c, pl.ds(s * rpt, rpt)])

    return pl.kernel(
        body,
        out_type=jax.ShapeDtypeStruct((NC, n_pad, f), F32),
        mesh=_sc_mesh(),
        scratch_types=[
            pltpu.VMEM((2, NBUF, 2, CHUNK), jnp.int32),
            pltpu.VMEM((NBUF, CHUNK, f), F32),
            pltpu.VMEM((zr, f), F32),
            pltpu.VMEM_SHARED((n_pad, f), F32),
        ] + [pltpu.SemaphoreType.DMA] * (5 * NBUF + 1),
    )


def _prep_body(degp_ref, x_ref, y_ref, dis_ref, sw_ref):
    deg = degp_ref[0] + degp_ref[1] + 1.0
    dis = lax.rsqrt(deg)
    dis_ref[...] = dis
    sw_ref[...] = 1.0 / deg
    y_ref[...] = x_ref[...] * dis


def _comb_body(p_ref, y_ref, sw_ref, o_ref):
    o_ref[...] = (p_ref[0] + p_ref[1] + y_ref[...]) * sw_ref[...]


def _run_prep(degp3, x_p, n_pad, f):
    grid = (n_pad // ROW_BLK,)
    return pl.pallas_call(
        _prep_body,
        grid=grid,
        in_specs=[
            pl.BlockSpec((NC, ROW_BLK, 1), lambda i: (0, i, 0)),
            pl.BlockSpec((ROW_BLK, f), lambda i: (i, 0)),
        ],
        out_specs=[
            pl.BlockSpec((ROW_BLK, f), lambda i: (i, 0)),
            pl.BlockSpec((ROW_BLK, 1), lambda i: (i, 0)),
            pl.BlockSpec((ROW_BLK, 1), lambda i: (i, 0)),
        ],
        out_shape=[
            jax.ShapeDtypeStruct((n_pad, f), F32),
            jax.ShapeDtypeStruct((n_pad, 1), F32),
            jax.ShapeDtypeStruct((n_pad, 1), F32),
        ],
    )(degp3, x_p)


def _run_comb(p, y, sw, n_pad, f):
    grid = (n_pad // ROW_BLK,)
    return pl.pallas_call(
        _comb_body,
        grid=grid,
        in_specs=[
            pl.BlockSpec((NC, ROW_BLK, f), lambda i: (0, i, 0)),
            pl.BlockSpec((ROW_BLK, f), lambda i: (i, 0)),
            pl.BlockSpec((ROW_BLK, 1), lambda i: (i, 0)),
        ],
        out_specs=pl.BlockSpec((ROW_BLK, f), lambda i: (i, 0)),
        out_shape=jax.ShapeDtypeStruct((n_pad, f), F32),
    )(p, y, sw)


def _make_mlp_body(n_classes):
    def body(p_ref, y_ref, dis_ref, w0_ref, b0_ref, w1_ref, b1_ref, w2_ref, b2_ref, o_ref):
        h = (p_ref[0] + p_ref[1] + y_ref[...]) * dis_ref[...]
        h = jnp.dot(h, w0_ref[...], preferred_element_type=F32) + b0_ref[...]
        h = jnp.maximum(h, 0.0)
        h = jnp.dot(h, w1_ref[...], preferred_element_type=F32) + b1_ref[...]
        h = jnp.maximum(h, 0.0)
        logits = jnp.dot(h, w2_ref[...], preferred_element_type=F32) + b2_ref[...]
        col = lax.broadcasted_iota(jnp.int32, logits.shape, 1)
        masked = jnp.where(col < n_classes, logits, jnp.float32(-1e30))
        m = jnp.max(masked, axis=1, keepdims=True)
        ssum = jnp.sum(jnp.exp(masked - m), axis=1, keepdims=True)
        o_ref[...] = logits - m - jnp.log(ssum)

    return body


def _run_mlp(p, y, dis, w0t, b0, w1t, b1, w2tp, b2p, n_pad, f, n_classes):
    h = w0t.shape[1]
    cp = w2tp.shape[1]
    grid = (n_pad // ROW_BLK,)
    return pl.pallas_call(
        _make_mlp_body(n_classes),
        grid=grid,
        in_specs=[
            pl.BlockSpec((NC, ROW_BLK, f), lambda i: (0, i, 0)),
            pl.BlockSpec((ROW_BLK, f), lambda i: (i, 0)),
            pl.BlockSpec((ROW_BLK, 1), lambda i: (i, 0)),
            pl.BlockSpec((f, h), lambda i: (0, 0)),
            pl.BlockSpec((h,), lambda i: (0,)),
            pl.BlockSpec((h, h), lambda i: (0, 0)),
            pl.BlockSpec((h,), lambda i: (0,)),
            pl.BlockSpec((h, cp), lambda i: (0, 0)),
            pl.BlockSpec((cp,), lambda i: (0,)),
        ],
        out_specs=pl.BlockSpec((ROW_BLK, cp), lambda i: (i, 0)),
        out_shape=jax.ShapeDtypeStruct((n_pad, cp), F32),
    )(p, y, dis, w0t, b0, w1t, b1, w2tp, b2p)


def kernel(x, edge_index, W0, b0, W1, b1, W2, b2):
    n_nodes, f = x.shape
    h_dim = W0.shape[0]
    n_classes = W2.shape[0]
    e = edge_index.shape[1]

    blk = NS * 16
    n_pad = -(-(n_nodes + 16) // blk) * blk
    n_pad = -(-n_pad // ROW_BLK) * ROW_BLK
    e_pad = -(-e // (NW * CHUNK * NBUF)) * (NW * CHUNK * NBUF)

    src = edge_index[0].astype(jnp.int32)
    dst = edge_index[1].astype(jnp.int32)
    npad_edges = e_pad - e
    ar = jnp.arange(npad_edges, dtype=jnp.int32)
    src_p = jnp.concatenate([src, (ar * 97) % n_nodes])
    dst_p = jnp.concatenate([dst, n_nodes + ar % (n_pad - n_nodes)])

    degp, pk = _make_deg_kernel(n_nodes, n_pad, e_pad)(src_p, dst_p)
    y, dis, sw = _run_prep(degp.reshape(NC, n_pad, 1), x.astype(F32),
                           n_pad, f)

    spmm = _make_spmm_kernel(n_pad, e_pad, f)
    for layer in range(3):
        p = spmm(y, pk)
        if layer < 2:
            y = _run_comb(p, y, sw, n_pad, f)

    cp = 128
    w2tp = jnp.zeros((h_dim, cp), F32).at[:, :n_classes].set(W2.T.astype(F32))
    b2p = jnp.zeros((cp,), F32).at[:n_classes].set(b2.astype(F32))
    out = _run_mlp(p, y, dis, W0.T.astype(F32), b0.astype(F32),
                   W1.T.astype(F32), b1.astype(F32), w2tp, b2p,
                   n_pad, f, n_classes)
    return out[:n_nodes, :n_classes]
